# baseline (speedup 1.0000x reference)
import functools

import jax
import jax.numpy as jnp
from jax import lax
from jax.experimental import pallas as pl
from jax.experimental.pallas import tpu as pltpu
from jax.experimental.pallas import tpu_sc as plsc

NUM_EMB = 100001
KERNEL_DIM = 64
INPUT_DIM = 16
BATCH = 16384

_NC = 2
_NS = 16
_NW = _NC * _NS
_B_PER_W = BATCH // _NW


def _gather_body(idx_hbm, table_hbm, out_hbm, idx_v, rows_v, sem):
    wid = lax.axis_index("s") * _NC + lax.axis_index("c")
    base = wid * _B_PER_W
    pltpu.sync_copy(idx_hbm.at[pl.ds(base, _B_PER_W)], idx_v)
    pltpu.async_copy(table_hbm.at[idx_v], rows_v, sem).wait()
    pltpu.sync_copy(rows_v, out_hbm.at[pl.ds(base, _B_PER_W)])


_sc_gather = functools.partial(
    pl.kernel,
    out_type=jax.ShapeDtypeStruct((BATCH, KERNEL_DIM), jnp.float32),
    mesh=plsc.VectorSubcoreMesh(
        core_axis_name="c", subcore_axis_name="s", num_cores=_NC, num_subcores=_NS
    ),
    scratch_types=[
        pltpu.VMEM((_B_PER_W,), jnp.int32),
        pltpu.VMEM((_B_PER_W, KERNEL_DIM), jnp.float32),
        pltpu.SemaphoreType.DMA,
    ],
    compiler_params=pltpu.CompilerParams(use_tc_tiling_on_sc=False),
)(_gather_body)


_BLK = 2048


def _tc_body(atom_ref, props_ref, wp_ref, bp_ref, wc1_ref, wc2_ref, bc_ref, out_ref):
    prop_out = (
        lax.dot_general(
            props_ref[...], wp_ref[...],
            (((1,), (1,)), ((), ())),
            preferred_element_type=jnp.float32,
        )
        + bp_ref[...]
    )
    atom_part = lax.dot_general(
        atom_ref[...], wc1_ref[...],
        (((1,), (1,)), ((), ())),
        preferred_element_type=jnp.float32,
    )
    comp_part = lax.dot_general(
        prop_out, wc2_ref[...],
        (((1,), (1,)), ((), ())),
        preferred_element_type=jnp.float32,
    )
    out_ref[...] = atom_part + comp_part + bc_ref[...]


def _tc_fused(atom_out, props, W_prop, b_prop, Wc1, Wc2, b_comp):
    grid = BATCH // _BLK
    return pl.pallas_call(
        _tc_body,
        grid=(grid,),
        in_specs=[
            pl.BlockSpec((_BLK, KERNEL_DIM), lambda i: (i, 0)),
            pl.BlockSpec((_BLK, INPUT_DIM), lambda i: (i, 0)),
            pl.BlockSpec((KERNEL_DIM, INPUT_DIM), lambda i: (0, 0)),
            pl.BlockSpec((1, KERNEL_DIM), lambda i: (0, 0)),
            pl.BlockSpec((KERNEL_DIM, KERNEL_DIM), lambda i: (0, 0)),
            pl.BlockSpec((KERNEL_DIM, KERNEL_DIM), lambda i: (0, 0)),
            pl.BlockSpec((1, KERNEL_DIM), lambda i: (0, 0)),
        ],
        out_specs=pl.BlockSpec((_BLK, KERNEL_DIM), lambda i: (i, 0)),
        out_shape=jax.ShapeDtypeStruct((BATCH, KERNEL_DIM), jnp.float32),
    )(atom_out, props, W_prop, b_prop, Wc1, Wc2, b_comp)


def kernel(props, atom_type_ids, atom_table, W_prop, b_prop, W_comp, b_comp):
    ids = atom_type_ids.astype(jnp.int32)
    atom_out = _sc_gather(ids, atom_table)
    Wc1 = W_comp[:, :KERNEL_DIM]
    Wc2 = W_comp[:, KERNEL_DIM:]
    return _tc_fused(
        atom_out,
        props,
        W_prop,
        b_prop.reshape(1, KERNEL_DIM),
        Wc1,
        Wc2,
        b_comp.reshape(1, KERNEL_DIM),
    )


# speedup vs baseline: 1.1422x; 1.1422x over previous
import functools

import jax
import jax.numpy as jnp
from jax import lax
from jax.experimental import pallas as pl
from jax.experimental.pallas import tpu as pltpu
from jax.experimental.pallas import tpu_sc as plsc

NUM_EMB = 100001
KERNEL_DIM = 64
INPUT_DIM = 16
BATCH = 16384

_PAD_ROWS = 100008
_PAD_COLS = 128

_NC = 2
_NS = 16
_NW = _NC * _NS
_B_PER_W = BATCH // _NW


def _gather_body(idx_hbm, table_hbm, out_hbm, idx_v, rows_v, sem):
    wid = lax.axis_index("s") * _NC + lax.axis_index("c")
    base = wid * _B_PER_W
    pltpu.sync_copy(idx_hbm.at[pl.ds(base, _B_PER_W)], idx_v)
    pltpu.async_copy(table_hbm.at[idx_v], rows_v, sem).wait()
    pltpu.sync_copy(rows_v, out_hbm.at[pl.ds(base, _B_PER_W)])


_sc_gather = functools.partial(
    pl.kernel,
    out_type=jax.ShapeDtypeStruct((BATCH, _PAD_COLS), jnp.float32),
    mesh=plsc.VectorSubcoreMesh(
        core_axis_name="c", subcore_axis_name="s", num_cores=_NC, num_subcores=_NS
    ),
    scratch_types=[
        pltpu.VMEM((_B_PER_W,), jnp.int32),
        pltpu.VMEM((_B_PER_W, _PAD_COLS), jnp.float32),
        pltpu.SemaphoreType.DMA,
    ],
)(_gather_body)


_BLK = 2048


def _tc_body(atom_ref, props_ref, wp_ref, bp_ref, wc1_ref, wc2_ref, bc_ref, out_ref):
    prop_out = (
        lax.dot_general(
            props_ref[...], wp_ref[...],
            (((1,), (1,)), ((), ())),
            preferred_element_type=jnp.float32,
        )
        + bp_ref[...]
    )
    atom_part = lax.dot_general(
        atom_ref[:, :KERNEL_DIM], wc1_ref[...],
        (((1,), (1,)), ((), ())),
        preferred_element_type=jnp.float32,
    )
    comp_part = lax.dot_general(
        prop_out, wc2_ref[...],
        (((1,), (1,)), ((), ())),
        preferred_element_type=jnp.float32,
    )
    out_ref[...] = atom_part + comp_part + bc_ref[...]


def _tc_fused(atom_out, props, W_prop, b_prop, Wc1, Wc2, b_comp):
    grid = BATCH // _BLK
    return pl.pallas_call(
        _tc_body,
        grid=(grid,),
        in_specs=[
            pl.BlockSpec((_BLK, _PAD_COLS), lambda i: (i, 0)),
            pl.BlockSpec((_BLK, INPUT_DIM), lambda i: (i, 0)),
            pl.BlockSpec((KERNEL_DIM, INPUT_DIM), lambda i: (0, 0)),
            pl.BlockSpec((1, KERNEL_DIM), lambda i: (0, 0)),
            pl.BlockSpec((KERNEL_DIM, KERNEL_DIM), lambda i: (0, 0)),
            pl.BlockSpec((KERNEL_DIM, KERNEL_DIM), lambda i: (0, 0)),
            pl.BlockSpec((1, KERNEL_DIM), lambda i: (0, 0)),
        ],
        out_specs=pl.BlockSpec((_BLK, KERNEL_DIM), lambda i: (i, 0)),
        out_shape=jax.ShapeDtypeStruct((BATCH, KERNEL_DIM), jnp.float32),
    )(atom_out, props, W_prop, b_prop, Wc1, Wc2, b_comp)


def kernel(props, atom_type_ids, atom_table, W_prop, b_prop, W_comp, b_comp):
    ids = atom_type_ids.astype(jnp.int32)
    table_pad = jnp.pad(
        atom_table,
        ((0, _PAD_ROWS - NUM_EMB), (0, _PAD_COLS - KERNEL_DIM)),
    )
    atom_out = _sc_gather(ids, table_pad)
    Wc1 = W_comp[:, :KERNEL_DIM]
    Wc2 = W_comp[:, KERNEL_DIM:]
    return _tc_fused(
        atom_out,
        props,
        W_prop,
        b_prop.reshape(1, KERNEL_DIM),
        Wc1,
        Wc2,
        b_comp.reshape(1, KERNEL_DIM),
    )


# speedup vs baseline: 1.4216x; 1.2446x over previous
import functools

import jax
import jax.numpy as jnp
from jax import lax
from jax.experimental import pallas as pl
from jax.experimental.pallas import tpu as pltpu
from jax.experimental.pallas import tpu_sc as plsc

NUM_EMB = 100001
KERNEL_DIM = 64
INPUT_DIM = 16
BATCH = 16384

_RBLK = 2048
_NRB = 49
_PAD_ROWS = _RBLK * _NRB
_PAD_COLS = 128

_NC = 2
_NS = 16
_NW = _NC * _NS
_B_PER_W = BATCH // _NW


def _repack_body(tab_t_ref, out_ref):
    out_ref[:, :KERNEL_DIM] = tab_t_ref[...].T


def _repack(table_t):
    return pl.pallas_call(
        _repack_body,
        grid=(_NRB,),
        in_specs=[pl.BlockSpec((KERNEL_DIM, _RBLK), lambda i: (0, i))],
        out_specs=pl.BlockSpec((_RBLK, _PAD_COLS), lambda i: (i, 0)),
        out_shape=jax.ShapeDtypeStruct((_PAD_ROWS, _PAD_COLS), jnp.float32),
    )(table_t)


def _gather_body(idx_hbm, table_hbm, out_hbm, idx_v, rows_v, sem):
    wid = lax.axis_index("s") * _NC + lax.axis_index("c")
    base = wid * _B_PER_W
    pltpu.sync_copy(idx_hbm.at[pl.ds(base, _B_PER_W)], idx_v)
    pltpu.async_copy(table_hbm.at[idx_v], rows_v, sem).wait()
    pltpu.sync_copy(rows_v, out_hbm.at[pl.ds(base, _B_PER_W)])


_sc_gather = functools.partial(
    pl.kernel,
    out_type=jax.ShapeDtypeStruct((BATCH, _PAD_COLS), jnp.float32),
    mesh=plsc.VectorSubcoreMesh(
        core_axis_name="c", subcore_axis_name="s", num_cores=_NC, num_subcores=_NS
    ),
    scratch_types=[
        pltpu.VMEM((_B_PER_W,), jnp.int32),
        pltpu.VMEM((_B_PER_W, _PAD_COLS), jnp.float32),
        pltpu.SemaphoreType.DMA,
    ],
)(_gather_body)


_BLK = 2048


def _tc_body(atom_ref, props_t_ref, wp_ref, bp_ref, wc1_ref, wc2_ref, bc_ref, out_ref):
    prop_out_t = (
        lax.dot_general(
            wp_ref[...], props_t_ref[...],
            (((1,), (0,)), ((), ())),
            preferred_element_type=jnp.float32,
        )
        + bp_ref[...]
    )
    atom_part_t = lax.dot_general(
        wc1_ref[...], atom_ref[:, :KERNEL_DIM],
        (((1,), (1,)), ((), ())),
        preferred_element_type=jnp.float32,
    )
    comp_part_t = lax.dot_general(
        wc2_ref[...], prop_out_t,
        (((1,), (0,)), ((), ())),
        preferred_element_type=jnp.float32,
    )
    out_ref[...] = atom_part_t + comp_part_t + bc_ref[...]


def _tc_fused(atom_out, props_t, W_prop, b_prop, Wc1, Wc2, b_comp):
    grid = BATCH // _BLK
    return pl.pallas_call(
        _tc_body,
        grid=(grid,),
        in_specs=[
            pl.BlockSpec((_BLK, _PAD_COLS), lambda i: (i, 0)),
            pl.BlockSpec((INPUT_DIM, _BLK), lambda i: (0, i)),
            pl.BlockSpec((KERNEL_DIM, INPUT_DIM), lambda i: (0, 0)),
            pl.BlockSpec((KERNEL_DIM, 1), lambda i: (0, 0)),
            pl.BlockSpec((KERNEL_DIM, KERNEL_DIM), lambda i: (0, 0)),
            pl.BlockSpec((KERNEL_DIM, KERNEL_DIM), lambda i: (0, 0)),
            pl.BlockSpec((KERNEL_DIM, 1), lambda i: (0, 0)),
        ],
        out_specs=pl.BlockSpec((KERNEL_DIM, _BLK), lambda i: (0, i)),
        out_shape=jax.ShapeDtypeStruct((KERNEL_DIM, BATCH), jnp.float32),
    )(atom_out, props_t, W_prop, b_prop, Wc1, Wc2, b_comp)


def kernel(props, atom_type_ids, atom_table, W_prop, b_prop, W_comp, b_comp):
    ids = atom_type_ids.astype(jnp.int32)
    table_pad = _repack(atom_table.T)
    atom_out = _sc_gather(ids, table_pad)
    Wc1 = W_comp[:, :KERNEL_DIM]
    Wc2 = W_comp[:, KERNEL_DIM:]
    out_t = _tc_fused(
        atom_out,
        props.T,
        W_prop,
        b_prop.reshape(KERNEL_DIM, 1),
        Wc1,
        Wc2,
        b_comp.reshape(KERNEL_DIM, 1),
    )
    return out_t.T


# speedup vs baseline: 1.6718x; 1.1760x over previous
import functools

import jax
import jax.numpy as jnp
from jax import lax
from jax.experimental import pallas as pl
from jax.experimental.pallas import tpu as pltpu
from jax.experimental.pallas import tpu_sc as plsc

NUM_EMB = 100001
KERNEL_DIM = 64
INPUT_DIM = 16
BATCH = 16384

_RBLK = 4096
_NRB = 25
_PAD_ROWS = _RBLK * _NRB
_PAD_COLS = 128

_NC = 2
_NS = 16
_NW = _NC * _NS
_B_PER_W = BATCH // _NW


def _repack_body(tab_t_ref, out_ref):
    t = tab_t_ref[...].T
    out_ref[...] = jnp.concatenate(
        [t, jnp.zeros((_RBLK, _PAD_COLS - KERNEL_DIM), jnp.float32)], axis=1
    )


def _repack(table_t):
    return pl.pallas_call(
        _repack_body,
        grid=(_NRB,),
        in_specs=[pl.BlockSpec((KERNEL_DIM, _RBLK), lambda i: (0, i))],
        out_specs=pl.BlockSpec((_RBLK, _PAD_COLS), lambda i: (i, 0)),
        out_shape=jax.ShapeDtypeStruct((_PAD_ROWS, _PAD_COLS), jnp.float32),
    )(table_t)


def _gather_body(idx_hbm, table_hbm, out_hbm, idx_v, rows_v, sem):
    wid = lax.axis_index("s") * _NC + lax.axis_index("c")
    base = wid * _B_PER_W
    pltpu.sync_copy(idx_hbm.at[pl.ds(base, _B_PER_W)], idx_v)
    pltpu.async_copy(table_hbm.at[idx_v], rows_v, sem).wait()
    pltpu.sync_copy(rows_v, out_hbm.at[pl.ds(base, _B_PER_W)])


_sc_gather = functools.partial(
    pl.kernel,
    out_type=jax.ShapeDtypeStruct((BATCH, _PAD_COLS), jnp.float32),
    mesh=plsc.VectorSubcoreMesh(
        core_axis_name="c", subcore_axis_name="s", num_cores=_NC, num_subcores=_NS
    ),
    scratch_types=[
        pltpu.VMEM((_B_PER_W,), jnp.int32),
        pltpu.VMEM((_B_PER_W, _PAD_COLS), jnp.float32),
        pltpu.SemaphoreType.DMA,
    ],
)(_gather_body)


_BLK = 2048


def _tc_body(atom_ref, props_t_ref, wp_ref, bp_ref, wc1_ref, wc2_ref, bc_ref, out_ref):
    prop_out_t = (
        lax.dot_general(
            wp_ref[...], props_t_ref[...],
            (((1,), (0,)), ((), ())),
            preferred_element_type=jnp.float32,
        )
        + bp_ref[...]
    )
    atom_part_t = lax.dot_general(
        wc1_ref[...], atom_ref[:, :KERNEL_DIM],
        (((1,), (1,)), ((), ())),
        preferred_element_type=jnp.float32,
    )
    comp_part_t = lax.dot_general(
        wc2_ref[...], prop_out_t,
        (((1,), (0,)), ((), ())),
        preferred_element_type=jnp.float32,
    )
    out_ref[...] = atom_part_t + comp_part_t + bc_ref[...]


def _tc_fused(atom_out, props_t, W_prop, b_prop, Wc1, Wc2, b_comp):
    grid = BATCH // _BLK
    return pl.pallas_call(
        _tc_body,
        grid=(grid,),
        in_specs=[
            pl.BlockSpec((_BLK, _PAD_COLS), lambda i: (i, 0)),
            pl.BlockSpec((INPUT_DIM, _BLK), lambda i: (0, i)),
            pl.BlockSpec((KERNEL_DIM, INPUT_DIM), lambda i: (0, 0)),
            pl.BlockSpec((KERNEL_DIM, 1), lambda i: (0, 0)),
            pl.BlockSpec((KERNEL_DIM, KERNEL_DIM), lambda i: (0, 0)),
            pl.BlockSpec((KERNEL_DIM, KERNEL_DIM), lambda i: (0, 0)),
            pl.BlockSpec((KERNEL_DIM, 1), lambda i: (0, 0)),
        ],
        out_specs=pl.BlockSpec((KERNEL_DIM, _BLK), lambda i: (0, i)),
        out_shape=jax.ShapeDtypeStruct((KERNEL_DIM, BATCH), jnp.float32),
    )(atom_out, props_t, W_prop, b_prop, Wc1, Wc2, b_comp)


def kernel(props, atom_type_ids, atom_table, W_prop, b_prop, W_comp, b_comp):
    ids = atom_type_ids.astype(jnp.int32)
    table_pad = _repack(atom_table.T)
    atom_out = _sc_gather(ids, table_pad)
    Wc1 = W_comp[:, :KERNEL_DIM]
    Wc2 = W_comp[:, KERNEL_DIM:]
    out_t = _tc_fused(
        atom_out,
        props.T,
        W_prop,
        b_prop.reshape(KERNEL_DIM, 1),
        Wc1,
        Wc2,
        b_comp.reshape(KERNEL_DIM, 1),
    )
    return out_t.T


# speedup vs baseline: 1.7140x; 1.0252x over previous
import functools

import jax
import jax.numpy as jnp
from jax import lax
from jax.experimental import pallas as pl
from jax.experimental.pallas import tpu as pltpu
from jax.experimental.pallas import tpu_sc as plsc

NUM_EMB = 100001
KERNEL_DIM = 64
INPUT_DIM = 16
BATCH = 16384

_RBLK = 4096
_NRB = 25
_PAD_ROWS = _RBLK * _NRB
_PAD_COLS = 128

_NC = 2
_NS = 16
_NW = _NC * _NS
_B_PER_W = BATCH // _NW


def _repack_body(tab_t_ref, out_ref):
    t = tab_t_ref[...].T
    out_ref[...] = jnp.concatenate(
        [t, jnp.zeros((_RBLK, _PAD_COLS - KERNEL_DIM), jnp.float32)], axis=1
    )


def _repack(table_t):
    return pl.pallas_call(
        _repack_body,
        grid=(_NRB,),
        in_specs=[pl.BlockSpec((KERNEL_DIM, _RBLK), lambda i: (0, i))],
        out_specs=pl.BlockSpec((_RBLK, _PAD_COLS), lambda i: (i, 0)),
        out_shape=jax.ShapeDtypeStruct((_PAD_ROWS, _PAD_COLS), jnp.float32),
    )(table_t)


def _gather_body(idx_hbm, table_hbm, out_hbm, idx_v, rows_v, sem):
    wid = lax.axis_index("s") * _NC + lax.axis_index("c")
    base = wid * _B_PER_W
    pltpu.sync_copy(idx_hbm.at[pl.ds(base, _B_PER_W)], idx_v)
    pltpu.async_copy(table_hbm.at[idx_v], rows_v, sem).wait()
    pltpu.sync_copy(rows_v, out_hbm.at[pl.ds(base, _B_PER_W)])


_sc_gather = functools.partial(
    pl.kernel,
    out_type=jax.ShapeDtypeStruct((BATCH, _PAD_COLS), jnp.float32),
    mesh=plsc.VectorSubcoreMesh(
        core_axis_name="c", subcore_axis_name="s", num_cores=_NC, num_subcores=_NS
    ),
    scratch_types=[
        pltpu.VMEM((_B_PER_W,), jnp.int32),
        pltpu.VMEM((_B_PER_W, _PAD_COLS), jnp.float32),
        pltpu.SemaphoreType.DMA,
    ],
)(_gather_body)


_BLK = 4096


def _tc_body(atom_ref, props_t_ref, wp_ref, bp_ref, wc1_ref, wc2_ref, bc_ref, out_ref):
    prop_out_t = (
        lax.dot_general(
            wp_ref[...], props_t_ref[...],
            (((1,), (0,)), ((), ())),
            preferred_element_type=jnp.float32,
        )
        + bp_ref[...]
    )
    atom_part_t = lax.dot_general(
        wc1_ref[...], atom_ref[:, :KERNEL_DIM],
        (((1,), (1,)), ((), ())),
        preferred_element_type=jnp.float32,
    )
    comp_part_t = lax.dot_general(
        wc2_ref[...], prop_out_t,
        (((1,), (0,)), ((), ())),
        preferred_element_type=jnp.float32,
    )
    out_ref[...] = atom_part_t + comp_part_t + bc_ref[...]


def _tc_fused(atom_out, props_t, W_prop, b_prop, Wc1, Wc2, b_comp):
    grid = BATCH // _BLK
    return pl.pallas_call(
        _tc_body,
        grid=(grid,),
        in_specs=[
            pl.BlockSpec((_BLK, _PAD_COLS), lambda i: (i, 0)),
            pl.BlockSpec((INPUT_DIM, _BLK), lambda i: (0, i)),
            pl.BlockSpec((KERNEL_DIM, INPUT_DIM), lambda i: (0, 0)),
            pl.BlockSpec((KERNEL_DIM, 1), lambda i: (0, 0)),
            pl.BlockSpec((KERNEL_DIM, KERNEL_DIM), lambda i: (0, 0)),
            pl.BlockSpec((KERNEL_DIM, KERNEL_DIM), lambda i: (0, 0)),
            pl.BlockSpec((KERNEL_DIM, 1), lambda i: (0, 0)),
        ],
        out_specs=pl.BlockSpec((KERNEL_DIM, _BLK), lambda i: (0, i)),
        out_shape=jax.ShapeDtypeStruct((KERNEL_DIM, BATCH), jnp.float32),
    )(atom_out, props_t, W_prop, b_prop, Wc1, Wc2, b_comp)


def kernel(props, atom_type_ids, atom_table, W_prop, b_prop, W_comp, b_comp):
    ids = atom_type_ids.astype(jnp.int32)
    table_pad = _repack(atom_table.T)
    atom_out = _sc_gather(ids, table_pad)
    Wc1 = W_comp[:, :KERNEL_DIM]
    Wc2 = W_comp[:, KERNEL_DIM:]
    out_t = _tc_fused(
        atom_out,
        props.T,
        W_prop,
        b_prop.reshape(KERNEL_DIM, 1),
        Wc1,
        Wc2,
        b_comp.reshape(KERNEL_DIM, 1),
    )
    return out_t.T
